# baseline (speedup 1.0000x reference)
import functools

import jax
import jax.numpy as jnp
from jax import lax
from jax.experimental import pallas as pl
from jax.experimental.pallas import tpu as pltpu
from jax.experimental.pallas import tpu_sc as plsc

_NUM_CLASSES = 1000
_BATCH = 16384
_NC = 2
_NS = 16
_NW = _NC * _NS
_COLS_PER_W = _BATCH // _NW
_CCHUNK = 120
_CSIZES = [120] * 8 + [40]
_N_CHUNKS = len(_CSIZES)
_GROUPS = _COLS_PER_W // 16


def _zero_buf(buf):
    zeros = jnp.zeros((16,), jnp.int32)

    def zero_body(r, carry):
        for g in range(_GROUPS):
            buf[r, pl.ds(g * 16, 16)] = zeros
        return carry

    lax.fori_loop(0, _CCHUNK, zero_body, 0)


def _sc_body(label_hbm, out_hbm, label_v, buf_a, buf_b, lsem, sem_a, sem_b):
    wid = lax.axis_index("s") * _NC + lax.axis_index("c")
    col0 = wid * _COLS_PER_W
    lcopy = pltpu.make_async_copy(
        label_hbm.at[pl.ds(col0, _COLS_PER_W)], label_v, lsem
    )
    lcopy.start()
    _zero_buf(buf_a)
    lcopy.wait()

    zeros = jnp.zeros((16,), jnp.int32)
    ones = jnp.ones((16,), jnp.int32)
    iota = lax.iota(jnp.int32, 16)
    bufs = (buf_a, buf_b)
    sems = (sem_a, sem_b)

    def set_chunk(buf, c0, size, value):
        out = []
        for g in range(_GROUPS):
            lv = label_v[pl.ds(g * 16, 16)]
            row = lv - c0
            col = g * 16 + iota
            mask = (lv >= c0) & (lv < c0 + size)
            out.append((row, col, mask))
            plsc.store_scatter(buf, [row, col], value, mask=mask)
        return out

    prev = [None, None]
    copies = [None, None]
    c0 = 0
    for c, size in enumerate(_CSIZES):
        b = c % 2
        buf = bufs[b]
        if copies[b] is not None:
            copies[b].wait()
            for row, col, mask in prev[b]:
                plsc.store_scatter(buf, [row, col], zeros, mask=mask)
        prev[b] = set_chunk(buf, c0, size, ones)
        cp = pltpu.make_async_copy(
            buf.at[pl.ds(0, size), pl.ds(0, _COLS_PER_W)],
            out_hbm.at[pl.ds(c0, size), pl.ds(col0, _COLS_PER_W)],
            sems[b],
        )
        cp.start()
        copies[b] = cp
        c0 += size
        if c == 0:
            _zero_buf(buf_b)
    copies[(_N_CHUNKS - 2) % 2].wait()
    copies[(_N_CHUNKS - 1) % 2].wait()


_one_hot_sc_t = functools.partial(
    pl.kernel,
    out_type=jax.ShapeDtypeStruct((_NUM_CLASSES, _BATCH), jnp.int32),
    mesh=plsc.VectorSubcoreMesh(core_axis_name="c", subcore_axis_name="s"),
    compiler_params=pltpu.CompilerParams(needs_layout_passes=False),
    scratch_types=[
        pltpu.VMEM((_COLS_PER_W,), jnp.int32),
        pltpu.VMEM((_CCHUNK, _COLS_PER_W), jnp.int32),
        pltpu.VMEM((_CCHUNK, _COLS_PER_W), jnp.int32),
        pltpu.SemaphoreType.DMA,
        pltpu.SemaphoreType.DMA,
        pltpu.SemaphoreType.DMA,
    ],
)(_sc_body)


def kernel(label):
    return _one_hot_sc_t(label).T


# speedup vs baseline: 1.0227x; 1.0227x over previous
import functools

import jax
import jax.numpy as jnp
from jax import lax
from jax.experimental import pallas as pl
from jax.experimental.pallas import tpu as pltpu
from jax.experimental.pallas import tpu_sc as plsc

_NUM_CLASSES = 1000
_BATCH = 16384
_NC = 2
_NS = 16
_NW = _NC * _NS
_COLS_PER_W = _BATCH // _NW
_CCHUNK = 200
_N_CHUNKS = _NUM_CLASSES // _CCHUNK
_GROUPS = _COLS_PER_W // 16
_BUF_WORDS = _CCHUNK * _COLS_PER_W


def _sc_body(label_hbm, out_hbm, label_v, buf_v, lsem):
    wid = lax.axis_index("s") * _NC + lax.axis_index("c")
    col0 = wid * _COLS_PER_W
    lcopy = pltpu.make_async_copy(
        label_hbm.at[pl.ds(col0, _COLS_PER_W)], label_v, lsem
    )
    lcopy.start()

    zeros = jnp.zeros((16,), jnp.int32)
    ones = jnp.ones((16,), jnp.int32)
    iota = lax.iota(jnp.int32, 16)

    def zero_body(r, carry):
        for g in range(_GROUPS):
            buf_v[r, pl.ds(g * 16, 16)] = zeros
        return carry

    lax.fori_loop(0, _CCHUNK, zero_body, 0)
    lcopy.wait()

    for c in range(_N_CHUNKS):
        c0 = c * _CCHUNK
        idxs = []
        for g in range(_GROUPS):
            lv = label_v[pl.ds(g * 16, 16)]
            row = lv - c0
            col = g * 16 + iota
            mask = (lv >= c0) & (lv < c0 + _CCHUNK)
            idxs.append((row, col, mask))
            plsc.store_scatter(buf_v, [row, col], ones, mask=mask)
        pltpu.sync_copy(
            buf_v,
            out_hbm.at[pl.ds(c0, _CCHUNK), pl.ds(col0, _COLS_PER_W)],
        )
        if c < _N_CHUNKS - 1:
            for row, col, mask in idxs:
                plsc.store_scatter(buf_v, [row, col], zeros, mask=mask)


_one_hot_sc_t = functools.partial(
    pl.kernel,
    out_type=jax.ShapeDtypeStruct((_NUM_CLASSES, _BATCH), jnp.int32),
    mesh=plsc.VectorSubcoreMesh(core_axis_name="c", subcore_axis_name="s"),
    compiler_params=pltpu.CompilerParams(needs_layout_passes=False),
    scratch_types=[
        pltpu.VMEM((_COLS_PER_W,), jnp.int32),
        pltpu.VMEM((_CCHUNK, _COLS_PER_W), jnp.int32),
        pltpu.SemaphoreType.DMA,
    ],
)(_sc_body)


def kernel(label):
    return _one_hot_sc_t(label).T


# speedup vs baseline: 1.0240x; 1.0013x over previous
import functools

import jax
import jax.numpy as jnp
from jax import lax
from jax.experimental import pallas as pl
from jax.experimental.pallas import tpu as pltpu
from jax.experimental.pallas import tpu_sc as plsc

_NUM_CLASSES = 1000
_BATCH = 16384
_NC = 2
_NS = 16
_NW = _NC * _NS
_COLS_PER_W = _BATCH // _NW
_CCHUNK = 200
_N_CHUNKS = _NUM_CLASSES // _CCHUNK
_GROUPS = _COLS_PER_W // 16
_SPLIT = 96
_BUF_WORDS = _CCHUNK * _COLS_PER_W


def _sc_body(label_hbm, out_hbm, label_v, buf_v, lsem):
    wid = lax.axis_index("s") * _NC + lax.axis_index("c")
    col0 = wid * _COLS_PER_W
    lcopy = pltpu.make_async_copy(
        label_hbm.at[pl.ds(col0, _COLS_PER_W)], label_v, lsem
    )
    lcopy.start()

    zeros = jnp.zeros((16,), jnp.int32)
    ones = jnp.ones((16,), jnp.int32)
    iota = lax.iota(jnp.int32, 16)

    def zero_rows(lo, hi):
        def zero_body(r, carry):
            for g in range(_GROUPS):
                buf_v[r, pl.ds(g * 16, 16)] = zeros
            return carry

        lax.fori_loop(lo, hi, zero_body, 0)

    def set_classes(lo, hi, value):
        out = []
        for g in range(_GROUPS):
            lv = label_v[pl.ds(g * 16, 16)]
            col = g * 16 + iota
            mask = (lv >= lo) & (lv < hi)
            out.append((lv, col, mask))
            plsc.store_scatter(buf_v, [lv - (lo // _CCHUNK) * _CCHUNK, col],
                               value, mask=mask)
        return out

    zero_rows(0, _SPLIT)
    lcopy.wait()
    idxs0a = set_classes(0, _SPLIT, ones)
    cp_a = pltpu.make_async_copy(
        buf_v.at[pl.ds(0, _SPLIT), pl.ds(0, _COLS_PER_W)],
        out_hbm.at[pl.ds(0, _SPLIT), pl.ds(col0, _COLS_PER_W)],
        lsem,
    )
    cp_a.start()
    zero_rows(_SPLIT, _CCHUNK)
    idxs0b = set_classes(_SPLIT, _CCHUNK, ones)
    cp_b = pltpu.make_async_copy(
        buf_v.at[pl.ds(_SPLIT, _CCHUNK - _SPLIT), pl.ds(0, _COLS_PER_W)],
        out_hbm.at[pl.ds(_SPLIT, _CCHUNK - _SPLIT), pl.ds(col0, _COLS_PER_W)],
        lsem,
    )
    cp_b.start()
    cp_a.wait()
    cp_b.wait()
    for row, col, mask in idxs0a + idxs0b:
        plsc.store_scatter(buf_v, [row, col], zeros, mask=mask)

    for c in range(1, _N_CHUNKS):
        c0 = c * _CCHUNK
        idxs = []
        for g in range(_GROUPS):
            lv = label_v[pl.ds(g * 16, 16)]
            row = lv - c0
            col = g * 16 + iota
            mask = (lv >= c0) & (lv < c0 + _CCHUNK)
            idxs.append((row, col, mask))
            plsc.store_scatter(buf_v, [row, col], ones, mask=mask)
        pltpu.sync_copy(
            buf_v,
            out_hbm.at[pl.ds(c0, _CCHUNK), pl.ds(col0, _COLS_PER_W)],
        )
        if c < _N_CHUNKS - 1:
            for row, col, mask in idxs:
                plsc.store_scatter(buf_v, [row, col], zeros, mask=mask)


_one_hot_sc_t = functools.partial(
    pl.kernel,
    out_type=jax.ShapeDtypeStruct((_NUM_CLASSES, _BATCH), jnp.int32),
    mesh=plsc.VectorSubcoreMesh(core_axis_name="c", subcore_axis_name="s"),
    compiler_params=pltpu.CompilerParams(needs_layout_passes=False),
    scratch_types=[
        pltpu.VMEM((_COLS_PER_W,), jnp.int32),
        pltpu.VMEM((_CCHUNK, _COLS_PER_W), jnp.int32),
        pltpu.SemaphoreType.DMA,
    ],
)(_sc_body)


def kernel(label):
    return _one_hot_sc_t(label).T
